# baseline (speedup 1.0000x reference)
import jax
import jax.numpy as jnp
from jax import lax
from jax.experimental import pallas as pl
from jax.experimental.pallas import tpu as pltpu
from jax.experimental.pallas import tpu_sc as plsc

ROWS = 16384
COLS = 2048
LANES_SC = 16
SUB = COLS // LANES_SC
ELEMS = ROWS * COLS
PER_SC = ELEMS // 2
TILE_CHUNK = 32768
SC_CHUNK = TILE_CHUNK * 16
N_CHUNKS = PER_SC // SC_CHUNK
PAIRS = (TILE_CHUNK // COLS) * (SUB // 2)


def _sc_flip(in_hbm, out_hbm, sp0, sp1, tv0, tv1,
             sdin0, sdin1, sout0, sout1):
    cc = lax.axis_index("c")
    ss = lax.axis_index("s")
    sc_base = cc * PER_SC
    sp = (sp0, sp1)
    tv = (tv0, tv1)
    sdin = (sdin0, sdin1)
    souts = (sout0, sout1)
    iota = lax.iota(jnp.int32, LANES_SC)

    def sc_off(ci):
        return sc_base + ci * SC_CHUNK

    def my_off(ci):
        return sc_off(ci) + ss * TILE_CHUNK

    @pl.when(ss == 0)
    def _():
        pltpu.async_copy(in_hbm.at[pl.ds(sc_off(0), SC_CHUNK)], sp0, sdin0)

    def outer(g, carry):
        for b in range(2):
            ci = 2 * g + b
            nb = 1 - b

            plsc.subcore_barrier()

            @pl.when((ss == 0) & (ci + 1 < N_CHUNKS))
            def _():
                pltpu.async_copy(
                    in_hbm.at[pl.ds(sc_off(ci + 1), SC_CHUNK)],
                    sp[nb], sdin[nb],
                )

            @pl.when(ss == 0)
            def _():
                pltpu.make_async_copy(
                    in_hbm.at[pl.ds(sc_off(ci), SC_CHUNK)], sp[b], sdin[b]
                ).wait()

            plsc.subcore_barrier()

            @pl.when(ci >= 2)
            def _():
                pltpu.make_async_copy(
                    tv[b],
                    out_hbm.at[pl.ds(my_off(ci - 2), TILE_CHUNK)],
                    souts[b],
                ).wait()

            pltpu.sync_copy(
                sp[b].at[pl.ds(ss * TILE_CHUNK, TILE_CHUNK)], tv[b]
            )

            @plsc.parallel_loop(0, PAIRS, unroll=8)
            def _(j):
                r = j >> 6
                k = j & (SUB // 2 - 1)
                a = (r << 11) + (k << 4)
                bo = (r << 11) + ((SUB - 1 - k) << 4)
                x = plsc.load_gather(tv[b], [(bo + LANES_SC - 1) - iota])
                y = plsc.load_gather(tv[b], [(a + LANES_SC - 1) - iota])
                tv[b][pl.ds(a, LANES_SC)] = x
                tv[b][pl.ds(bo, LANES_SC)] = y

            pltpu.async_copy(
                tv[b], out_hbm.at[pl.ds(my_off(ci), TILE_CHUNK)], souts[b]
            )
        return carry

    lax.fori_loop(0, N_CHUNKS // 2, outer, 0)

    pltpu.make_async_copy(
        tv0, out_hbm.at[pl.ds(my_off(N_CHUNKS - 2), TILE_CHUNK)], sout0
    ).wait()
    pltpu.make_async_copy(
        tv1, out_hbm.at[pl.ds(my_off(N_CHUNKS - 1), TILE_CHUNK)], sout1
    ).wait()


def kernel(inputs, cond_inputs):
    flat_in = inputs.reshape(ELEMS)
    mesh = plsc.VectorSubcoreMesh(core_axis_name="c", subcore_axis_name="s")
    f = pl.kernel(
        _sc_flip,
        mesh=mesh,
        out_type=jax.ShapeDtypeStruct((ELEMS,), jnp.float32),
        compiler_params=pltpu.CompilerParams(needs_layout_passes=False),
        scratch_types=[
            pltpu.VMEM_SHARED((SC_CHUNK,), jnp.float32),
            pltpu.VMEM_SHARED((SC_CHUNK,), jnp.float32),
            pltpu.VMEM((TILE_CHUNK,), jnp.float32),
            pltpu.VMEM((TILE_CHUNK,), jnp.float32),
            pltpu.SemaphoreType.DMA,
            pltpu.SemaphoreType.DMA,
            pltpu.SemaphoreType.DMA,
            pltpu.SemaphoreType.DMA,
        ],
    )
    out = f(flat_in)
    return (out.reshape(ROWS, COLS), 0.0)


# speedup vs baseline: 4.0754x; 4.0754x over previous
import jax
import jax.numpy as jnp
from jax.experimental import pallas as pl

ROWS = 16384
COLS = 2048
BLOCK_ROWS = 1024
LANES = 128
NUM_SUB = COLS // LANES


def _flip_block(in_ref, out_ref):
    rev = (LANES - 1) - jax.lax.broadcasted_iota(
        jnp.int32, (BLOCK_ROWS, LANES), 1
    )
    for j in range(NUM_SUB):
        src = NUM_SUB - 1 - j
        x = in_ref[:, src * LANES:(src + 1) * LANES]
        out_ref[:, j * LANES:(j + 1) * LANES] = jnp.take_along_axis(
            x, rev, axis=1
        )


def kernel(inputs, cond_inputs):
    out = pl.pallas_call(
        _flip_block,
        grid=(ROWS // BLOCK_ROWS,),
        in_specs=[pl.BlockSpec((BLOCK_ROWS, COLS), lambda i: (i, 0))],
        out_specs=pl.BlockSpec((BLOCK_ROWS, COLS), lambda i: (i, 0)),
        out_shape=jax.ShapeDtypeStruct((ROWS, COLS), inputs.dtype),
    )(inputs)
    return (out, 0.0)
